# baseline (speedup 1.0000x reference)
import functools

import numpy as np

import jax
import jax.numpy as jnp
from jax import lax
from jax.experimental import pallas as pl
from jax.experimental.pallas import tpu as pltpu
from jax.experimental.pallas import tpu_sc as plsc

NC = 2
NS = 16
NW = NC * NS
LANES = 16
CHUNK = 128


def _sc_mesh():
    return plsc.VectorSubcoreMesh(core_axis_name="c", subcore_axis_name="s")


def _make_deg_kernel(e, ch_per_w, n_pad):
    epw_pad = ch_per_w * CHUNK

    @functools.partial(
        pl.kernel,
        out_type=jax.ShapeDtypeStruct((NW, n_pad), jnp.float32),
        mesh=_sc_mesh(),
        compiler_params=pltpu.CompilerParams(needs_layout_passes=False),
        scratch_types=[
            pltpu.VMEM((epw_pad,), jnp.int32),
            pltpu.VMEM((n_pad,), jnp.float32),
        ],
    )
    def deg_kernel(rowp_hbm, out_hbm, idx_v, deg_v):
        c = lax.axis_index("c")
        s = lax.axis_index("s")
        wid = s * NC + c
        pltpu.sync_copy(rowp_hbm.at[pl.ds(wid * epw_pad, epw_pad)], idx_v)

        zeros16 = jnp.zeros((LANES,), jnp.float32)

        def zero_body(i, carry):
            deg_v[pl.ds(i * LANES, LANES)] = zeros16
            return carry

        lax.fori_loop(0, n_pad // LANES, zero_body, 0, unroll=4)

        ones16 = jnp.ones((LANES,), jnp.float32)
        lane = lax.iota(jnp.int32, LANES)
        base = wid * epw_pad

        def edge_body(p, carry):
            idx = idx_v[pl.ds(p * LANES, LANES)]
            msk = base + p * LANES + lane < e
            plsc.addupdate_scatter(deg_v, [idx], ones16, mask=msk)
            return carry

        lax.fori_loop(0, epw_pad // LANES, edge_body, 0, unroll=4)
        pltpu.sync_copy(deg_v, out_hbm.at[wid])

    return deg_kernel


NBUF = 2
NSEG = 2


def _make_agg_kernel(ch_per_w, n_pad, d):
    assert ch_per_w % (NSEG * NBUF) == 0
    ch_seg = ch_per_w // NSEG

    @functools.partial(
        pl.kernel,
        out_type=jax.ShapeDtypeStruct((NC, n_pad, d), jnp.float32),
        mesh=_sc_mesh(),
        compiler_params=pltpu.CompilerParams(needs_layout_passes=False),
        scratch_types=[
            pltpu.VMEM((ch_seg, CHUNK), jnp.int32),
            pltpu.VMEM((ch_seg, CHUNK), jnp.int32),
            [pltpu.VMEM((CHUNK, d), jnp.float32) for _ in range(NBUF)],
            pltpu.VMEM_SHARED((n_pad, d), jnp.float32),
            [pltpu.SemaphoreType.DMA for _ in range(NBUF)],
            pltpu.SemaphoreType.DMA,
        ],
    )
    def agg_kernel(y_hbm, edges_hbm, zeros_hbm, out_hbm,
                   col_v, row_v, bufs, acc_sh, sems, zsem):
        c = lax.axis_index("c")
        s = lax.axis_index("s")
        wid = s * NC + c
        rows_per_tile = n_pad // NS
        zcp = pltpu.async_copy(
            zeros_hbm.at[pl.ds(s * rows_per_tile, rows_per_tile)],
            acc_sh.at[pl.ds(s * rows_per_tile, rows_per_tile)],
            zsem,
        )
        def seg_body(seg, carry):
            pltpu.sync_copy(
                edges_hbm.at[1, wid, pl.ds(seg * ch_seg, ch_seg)], col_v)
            pltpu.sync_copy(
                edges_hbm.at[0, wid, pl.ds(seg * ch_seg, ch_seg)], row_v)
            for b in range(NBUF):
                pltpu.async_copy(y_hbm.at[col_v.at[b]], bufs[b], sems[b])

            @pl.when(seg == 0)
            def _():
                zcp.wait()
                plsc.subcore_barrier()

            def group_body(g, carry2):
                base = g * NBUF
                for b in range(NBUF):
                    j = base + b
                    pltpu.make_async_copy(
                        y_hbm.at[col_v.at[j]], bufs[b], sems[b]).wait()
                    pltpu.sync_copy(
                        bufs[b], acc_sh.at[row_v.at[j]], add=True)
                    nxt = j + NBUF

                    @pl.when(nxt < ch_seg)
                    def _():
                        pltpu.async_copy(
                            y_hbm.at[col_v.at[nxt]], bufs[b], sems[b])
                return carry2

            lax.fori_loop(0, ch_seg // NBUF, group_body, 0)
            return carry

        lax.fori_loop(0, NSEG, seg_body, 0)
        plsc.subcore_barrier()
        pltpu.sync_copy(
            acc_sh.at[pl.ds(s * rows_per_tile, rows_per_tile)],
            out_hbm.at[c, pl.ds(s * rows_per_tile, rows_per_tile)],
        )

    return agg_kernel


def _prep_body(degp_ref, x_ref, w_ref, y_ref, dis_ref):
    deg = jnp.sum(degp_ref[...], axis=0)
    dis = jnp.where(deg > 0.0, lax.rsqrt(deg), 0.0)
    dis_ref[...] = dis
    n = x_ref.shape[0]
    n_pad = y_ref.shape[0]
    z = lax.dot_general(
        x_ref[...], w_ref[...],
        (((1,), (1,)), ((), ())),
        preferred_element_type=jnp.float32,
    )
    y_ref[pl.ds(0, n), :] = dis[:n, None] * z
    y_ref[pl.ds(n, n_pad - n), :] = jnp.zeros(
        (n_pad - n, z.shape[1]), jnp.float32)


def _fin_body(s_ref, dis_ref, b_ref, o_ref):
    n = o_ref.shape[0]
    ssum = s_ref[0, pl.ds(0, n), :] + s_ref[1, pl.ds(0, n), :]
    val = dis_ref[...][:n, None] * ssum + b_ref[...]
    o_ref[...] = jnp.maximum(val, 0.0)


def kernel(x, edge_index, W, b):
    n, d_in = x.shape
    d_out = W.shape[0]
    e = edge_index.shape[1]

    ch_per_w = -(-e // (NW * CHUNK))
    ch_per_w = -(-ch_per_w // (NSEG * NBUF)) * (NSEG * NBUF)
    e_pad = NW * ch_per_w * CHUNK
    n_pad = -(-n // (NS * LANES)) * (NS * LANES)

    karr = jnp.arange(e_pad - e, dtype=jnp.int32)
    row_p = jnp.concatenate([edge_index[0], (karr * 37) % n_pad])
    col_p = jnp.concatenate([edge_index[1], n + (karr * 3) % (n_pad - n)])
    edges4 = jnp.stack([row_p, col_p]).reshape(2, NW, ch_per_w, CHUNK)

    zeros_nd = jnp.asarray(np.zeros((n_pad, d_in), np.float32))
    deg_parts = _make_deg_kernel(e, ch_per_w, n_pad)(row_p)

    y, dis = pl.pallas_call(
        _prep_body,
        out_shape=(
            jax.ShapeDtypeStruct((n_pad, d_in), jnp.float32),
            jax.ShapeDtypeStruct((n_pad,), jnp.float32),
        ),
    )(deg_parts, x, W)

    s_parts = _make_agg_kernel(ch_per_w, n_pad, d_in)(
        y, edges4, zeros_nd)

    out = pl.pallas_call(
        _fin_body,
        out_shape=jax.ShapeDtypeStruct((n, d_out), jnp.float32),
    )(s_parts, dis, b.reshape(1, d_out))
    return out


# speedup vs baseline: 1.0442x; 1.0442x over previous
import functools

import jax
import jax.numpy as jnp
from jax import lax
from jax.experimental import pallas as pl
from jax.experimental.pallas import tpu as pltpu
from jax.experimental.pallas import tpu_sc as plsc

NC = 2
NS = 16
NW = NC * NS
LANES = 16
CHUNK = 128


def _sc_mesh():
    return plsc.VectorSubcoreMesh(core_axis_name="c", subcore_axis_name="s")


def _make_deg_kernel(e, ch_per_w, n_pad):
    epw_pad = ch_per_w * CHUNK

    @functools.partial(
        pl.kernel,
        out_type=jax.ShapeDtypeStruct((NW, n_pad), jnp.float32),
        mesh=_sc_mesh(),
        compiler_params=pltpu.CompilerParams(needs_layout_passes=False),
        scratch_types=[
            pltpu.VMEM((ch_per_w, CHUNK), jnp.int32),
            pltpu.VMEM((n_pad,), jnp.float32),
        ],
    )
    def deg_kernel(edges_hbm, out_hbm, idx_v, deg_v):
        c = lax.axis_index("c")
        s = lax.axis_index("s")
        wid = s * NC + c
        pltpu.sync_copy(edges_hbm.at[0, wid], idx_v)

        zeros16 = jnp.zeros((LANES,), jnp.float32)

        def zero_body(i, carry):
            deg_v[pl.ds(i * LANES, LANES)] = zeros16
            return carry

        lax.fori_loop(0, n_pad // LANES, zero_body, 0, unroll=8)

        ones16 = jnp.ones((LANES,), jnp.float32)
        lane = lax.iota(jnp.int32, LANES)
        base = wid * epw_pad

        def edge_body(p, carry):
            j = p // (CHUNK // LANES)
            l = p % (CHUNK // LANES)
            idx = idx_v[j, pl.ds(l * LANES, LANES)]
            msk = base + p * LANES + lane < e
            plsc.addupdate_scatter(deg_v, [idx], ones16, mask=msk)
            return carry

        lax.fori_loop(0, epw_pad // LANES, edge_body, 0, unroll=8)
        pltpu.sync_copy(deg_v, out_hbm.at[wid])

    return deg_kernel


NBUF = 2
NSEG = 2


def _make_agg_kernel(ch_per_w, n_pad, d):
    assert ch_per_w % (NSEG * NBUF) == 0
    ch_seg = ch_per_w // NSEG

    @functools.partial(
        pl.kernel,
        out_type=jax.ShapeDtypeStruct((NC, n_pad, d), jnp.float32),
        mesh=_sc_mesh(),
        compiler_params=pltpu.CompilerParams(needs_layout_passes=False),
        scratch_types=[
            pltpu.VMEM((ch_seg, CHUNK), jnp.int32),
            pltpu.VMEM((ch_seg, CHUNK), jnp.int32),
            [pltpu.VMEM((CHUNK, d), jnp.float32) for _ in range(NBUF)],
            pltpu.VMEM_SHARED((n_pad, d), jnp.float32),
            [pltpu.SemaphoreType.DMA for _ in range(NBUF)],
            pltpu.SemaphoreType.DMA,
        ],
    )
    def agg_kernel(y_hbm, edges_hbm, zeros_hbm, out_hbm,
                   col_v, row_v, bufs, acc_sh, sems, zsem):
        c = lax.axis_index("c")
        s = lax.axis_index("s")
        wid = s * NC + c
        rows_per_tile = n_pad // NS
        zcp = pltpu.async_copy(
            zeros_hbm.at[pl.ds(s * rows_per_tile, rows_per_tile)],
            acc_sh.at[pl.ds(s * rows_per_tile, rows_per_tile)],
            zsem,
        )
        first = True
        for seg in range(NSEG):
            pltpu.sync_copy(
                edges_hbm.at[1, wid, pl.ds(seg * ch_seg, ch_seg)], col_v)
            pltpu.sync_copy(
                edges_hbm.at[0, wid, pl.ds(seg * ch_seg, ch_seg)], row_v)
            for b in range(NBUF):
                pltpu.async_copy(y_hbm.at[col_v.at[b]], bufs[b], sems[b])
            if first:
                first = False
                zcp.wait()
                plsc.subcore_barrier()

            def group_body(g, carry):
                base = g * NBUF
                for b in range(NBUF):
                    j = base + b
                    pltpu.make_async_copy(
                        y_hbm.at[col_v.at[j]], bufs[b], sems[b]).wait()
                    pltpu.sync_copy(
                        bufs[b], acc_sh.at[row_v.at[j]], add=True)
                    nxt = j + NBUF

                    @pl.when(nxt < ch_seg)
                    def _():
                        pltpu.async_copy(
                            y_hbm.at[col_v.at[nxt]], bufs[b], sems[b])
                return carry

            lax.fori_loop(0, ch_seg // NBUF, group_body, 0)
        plsc.subcore_barrier()
        pltpu.sync_copy(
            acc_sh.at[pl.ds(s * rows_per_tile, rows_per_tile)],
            out_hbm.at[c, pl.ds(s * rows_per_tile, rows_per_tile)],
        )

    return agg_kernel


def _prep_body(degp_ref, x_ref, w_ref, y_ref, dis_ref):
    deg = jnp.sum(degp_ref[...], axis=0)
    dis = jnp.where(deg > 0.0, lax.rsqrt(deg), 0.0)
    dis_ref[...] = dis
    n = x_ref.shape[0]
    n_pad = y_ref.shape[0]
    z = lax.dot_general(
        x_ref[...], w_ref[...],
        (((1,), (1,)), ((), ())),
        preferred_element_type=jnp.float32,
    )
    y_ref[pl.ds(0, n), :] = dis[:n, None] * z
    y_ref[pl.ds(n, n_pad - n), :] = jnp.zeros(
        (n_pad - n, z.shape[1]), jnp.float32)


def _fin_body(s_ref, dis_ref, b_ref, o_ref):
    n = o_ref.shape[0]
    ssum = s_ref[0, pl.ds(0, n), :] + s_ref[1, pl.ds(0, n), :]
    val = dis_ref[...][:n, None] * ssum + b_ref[...]
    o_ref[...] = jnp.maximum(val, 0.0)


def kernel(x, edge_index, W, b):
    n, d_in = x.shape
    d_out = W.shape[0]
    e = edge_index.shape[1]

    ch_per_w = -(-e // (NW * CHUNK))
    ch_per_w = -(-ch_per_w // (NSEG * NBUF)) * (NSEG * NBUF)
    e_pad = NW * ch_per_w * CHUNK
    n_pad = -(-n // (NS * LANES)) * (NS * LANES)

    karr = jnp.arange(e_pad - e, dtype=jnp.int32)
    pad2 = jnp.stack([
        (karr * 37) % n_pad,
        n + (karr * 3) % (n_pad - n),
    ])
    edges4 = jnp.concatenate([edge_index, pad2], axis=1).reshape(
        2, NW, ch_per_w, CHUNK)

    zeros_nd = jnp.zeros((n_pad, d_in), jnp.float32)
    deg_parts = _make_deg_kernel(e, ch_per_w, n_pad)(edges4)

    y, dis = pl.pallas_call(
        _prep_body,
        out_shape=(
            jax.ShapeDtypeStruct((n_pad, d_in), jnp.float32),
            jax.ShapeDtypeStruct((n_pad,), jnp.float32),
        ),
    )(deg_parts, x, W)

    s_parts = _make_agg_kernel(ch_per_w, n_pad, d_in)(
        y, edges4, zeros_nd)

    out = pl.pallas_call(
        _fin_body,
        out_shape=jax.ShapeDtypeStruct((n, d_out), jnp.float32),
    )(s_parts, dis, b.reshape(1, d_out))
    return out
